# Initial kernel scaffold; baseline (speedup 1.0000x reference)
import jax
import jax.numpy as jnp
from jax.experimental import pallas as pl


def kernel(x, edge_index, W_self0, W_neigh0, b0, W_self1, W_neigh1, b1, W_self2, W_neigh2, b2, gamma0, beta0, gamma1, beta1, W_lin, b_lin):
    raise NotImplementedError("write your pallas kernel here")


# speedup vs baseline: 1.1332x; 1.1332x over previous
import jax
import jax.numpy as jnp
from jax.experimental import pallas as pl

N = 10000


def _mean_agg(h, src, dst, deg):
    s = jax.ops.segment_sum(h[src], dst, num_segments=N)
    return s / jnp.maximum(deg, 1.0)[:, None]


def _bn(h, gamma, beta):
    mu = jnp.mean(h, axis=0)
    var = jnp.var(h, axis=0)
    return (h - mu) / jnp.sqrt(var + 1e-5) * gamma + beta


def _final_kernel(s2_ref, t2_ref, ws2_ref, wn2_ref, b2_ref, wlin_ref, blin_ref, out_ref):
    hg = (jnp.dot(s2_ref[...], ws2_ref[...], preferred_element_type=jnp.float32)
          + jnp.dot(t2_ref[...], wn2_ref[...], preferred_element_type=jnp.float32)
          + float(N) * b2_ref[...])
    out_ref[...] = jnp.dot(hg, wlin_ref[...], preferred_element_type=jnp.float32) + blin_ref[...]


def kernel(x, edge_index, W_self0, W_neigh0, b0, W_self1, W_neigh1, b1,
           W_self2, W_neigh2, b2, gamma0, beta0, gamma1, beta1, W_lin, b_lin):
    src = edge_index[0]
    dst = edge_index[1]
    ones = jnp.ones((src.shape[0],), dtype=jnp.float32)
    deg = jax.ops.segment_sum(ones, dst, num_segments=N)

    hn0 = _mean_agg(x, src, dst, deg)
    h1 = jax.nn.relu(_bn(x @ W_self0 + hn0 @ W_neigh0 + b0, gamma0, beta0))
    hn1 = _mean_agg(h1, src, dst, deg)
    h2 = jax.nn.relu(_bn(h1 @ W_self1 + hn1 @ W_neigh1 + b1, gamma1, beta1))

    w_e = 1.0 / jnp.maximum(deg, 1.0)[dst]
    c = jax.ops.segment_sum(w_e, src, num_segments=N)
    s2 = jnp.sum(h2, axis=0, keepdims=True)
    t2 = (c[None, :] @ h2)

    out = pl.pallas_call(
        _final_kernel,
        out_shape=jax.ShapeDtypeStruct((1, W_lin.shape[1]), jnp.float32),
    )(s2, t2, W_self2, W_neigh2, b2[None, :], W_lin, b_lin[None, :])
    return out


# speedup vs baseline: 3.6516x; 3.2223x over previous
import functools

import jax
import jax.numpy as jnp
from jax import lax
from jax.experimental import pallas as pl
from jax.experimental.pallas import tpu as pltpu
from jax.experimental.pallas import tpu_sc as plsc

N = 10000
E = 320000
D_IN = 128
D_H = 256
D_OUT = 64

NC = 2
NS = 16
K = 128
EROWS = 2560
E_PAD = EROWS * K
N_ROWS = 10240
ROWS_PER_TILE = N_ROWS // NS
N_DEG = 10240
DEG_PER_TILE = N_DEG // NS
A_ROWS = EROWS // (NC * NS)
C_ROWS = EROWS // NS
IB = 16

_mesh = plsc.VectorSubcoreMesh(core_axis_name="c", subcore_axis_name="s")


def _zero_accs(zfeat, zdeg, acc, dacc, s):
    pltpu.sync_copy(zfeat, acc.at[pl.ds(s * ROWS_PER_TILE, ROWS_PER_TILE)])
    pltpu.sync_copy(zdeg, dacc.at[pl.ds(s * DEG_PER_TILE, DEG_PER_TILE)])


def _fill_ones(ones_v):
    for j in range(K // 16):
        ones_v[pl.ds(j * 16, 16)] = jnp.ones((16,), jnp.float32)


def _agg1_body(x_hbm, srcm, dstm, zfeat, zdeg,
               s0a, s0b, dega, degb,
               src_v, dst_v, rows_v, ones_v, sem,
               acc, dacc):
    c = lax.axis_index("c")
    s = lax.axis_index("s")
    wid = c * NS + s
    _zero_accs(zfeat, zdeg, acc, dacc, s)
    base = wid * A_ROWS
    _fill_ones(ones_v)
    plsc.subcore_barrier()

    def blk(bi, carry):
        pltpu.sync_copy(srcm.at[pl.ds(base + bi * IB, IB)], src_v)
        pltpu.sync_copy(dstm.at[pl.ds(base + bi * IB, IB)], dst_v)

        def body(j, carry2):
            pltpu.async_copy(x_hbm.at[src_v.at[j]], rows_v, sem).wait()
            pltpu.sync_copy(rows_v, acc.at[dst_v.at[j]], add=True)
            pltpu.sync_copy(ones_v, dacc.at[dst_v.at[j]], add=True)
            return carry2

        return lax.fori_loop(0, IB, body, carry)

    lax.fori_loop(0, A_ROWS // IB, blk, 0)
    plsc.subcore_barrier()

    rsl = pl.ds(s * ROWS_PER_TILE, ROWS_PER_TILE)
    dsl = pl.ds(s * DEG_PER_TILE, DEG_PER_TILE)

    @pl.when(c == 0)
    def _():
        pltpu.sync_copy(acc.at[rsl], s0a.at[rsl])
        pltpu.sync_copy(dacc.at[dsl], dega.at[dsl])

    @pl.when(c == 1)
    def _():
        pltpu.sync_copy(acc.at[rsl], s0b.at[rsl])
        pltpu.sync_copy(dacc.at[dsl], degb.at[dsl])


_agg1 = functools.partial(
    pl.kernel,
    out_type=[
        jax.ShapeDtypeStruct((N_ROWS, D_IN), jnp.float32),
        jax.ShapeDtypeStruct((N_ROWS, D_IN), jnp.float32),
        jax.ShapeDtypeStruct((N_DEG,), jnp.float32),
        jax.ShapeDtypeStruct((N_DEG,), jnp.float32),
    ],
    mesh=_mesh,
    scratch_types=[
        pltpu.VMEM((IB, K), jnp.int32),
        pltpu.VMEM((IB, K), jnp.int32),
        pltpu.VMEM((K, D_IN), jnp.float32),
        pltpu.VMEM((K,), jnp.float32),
        pltpu.SemaphoreType.DMA,
        pltpu.VMEM_SHARED((N_ROWS, D_IN), jnp.float32),
        pltpu.VMEM_SHARED((N_DEG,), jnp.float32),
    ],
)(_agg1_body)


def _agg2_body(h1a, h1b, deg_hbm, srcm, dstm, zfeat, zdeg,
               s1a, s1b, cout,
               src_v, dst_v, rows_v, degv, wv, sem,
               acc, cacc):
    c = lax.axis_index("c")
    s = lax.axis_index("s")
    _zero_accs(zfeat, zdeg, acc, cacc, s)
    base = s * C_ROWS
    plsc.subcore_barrier()

    def blk(bi, carry):
        pltpu.sync_copy(srcm.at[pl.ds(base + bi * IB, IB)], src_v)
        pltpu.sync_copy(dstm.at[pl.ds(base + bi * IB, IB)], dst_v)

        def body(j, carry2):
            @pl.when(c == 0)
            def _():
                pltpu.async_copy(h1a.at[src_v.at[j]], rows_v, sem).wait()

            @pl.when(c == 1)
            def _():
                pltpu.async_copy(h1b.at[src_v.at[j]], rows_v, sem).wait()

            pltpu.sync_copy(rows_v, acc.at[dst_v.at[j]], add=True)

            @pl.when(c == 0)
            def _():
                pltpu.async_copy(deg_hbm.at[dst_v.at[j]], degv, sem).wait()
                eid0 = (base + bi * IB + j) * K
                for u in range(K // 16):
                    d16 = degv[pl.ds(u * 16, 16)]
                    eid = eid0 + u * 16 + lax.iota(jnp.int32, 16)
                    w16 = jnp.where(eid < E, 1.0 / jnp.maximum(d16, 1.0), 0.0)
                    wv[pl.ds(u * 16, 16)] = w16
                pltpu.sync_copy(wv, cacc.at[src_v.at[j]], add=True)

            return carry2

        return lax.fori_loop(0, IB, body, carry)

    lax.fori_loop(0, C_ROWS // IB, blk, 0)
    plsc.subcore_barrier()

    rsl = pl.ds(s * ROWS_PER_TILE, ROWS_PER_TILE)
    dsl = pl.ds(s * DEG_PER_TILE, DEG_PER_TILE)

    @pl.when(c == 0)
    def _():
        pltpu.sync_copy(acc.at[rsl], s1a.at[rsl])
        pltpu.sync_copy(cacc.at[dsl], cout.at[dsl])

    @pl.when(c == 1)
    def _():
        pltpu.sync_copy(acc.at[rsl], s1b.at[rsl])


_agg2 = functools.partial(
    pl.kernel,
    out_type=[
        jax.ShapeDtypeStruct((N_ROWS, D_IN), jnp.float32),
        jax.ShapeDtypeStruct((N_ROWS, D_IN), jnp.float32),
        jax.ShapeDtypeStruct((N_DEG,), jnp.float32),
    ],
    mesh=_mesh,
    scratch_types=[
        pltpu.VMEM((IB, K), jnp.int32),
        pltpu.VMEM((IB, K), jnp.int32),
        pltpu.VMEM((K, D_IN), jnp.float32),
        pltpu.VMEM((K,), jnp.float32),
        pltpu.VMEM((K,), jnp.float32),
        pltpu.SemaphoreType.DMA,
        pltpu.VMEM_SHARED((N_ROWS, D_IN), jnp.float32),
        pltpu.VMEM_SHARED((N_DEG,), jnp.float32),
    ],
)(_agg2_body)


RB = 400
GRID = N // RB


def _dense1_body(x_ref, sa_ref, sb_ref, deg_ref, ws_ref, wn_ref, b_ref,
                 z_ref, sum_ref, sq_ref):
    i = pl.program_id(0)
    r = 1.0 / jnp.maximum(deg_ref[...], 1.0)
    hn = (sa_ref[...] + sb_ref[...]) * r
    z = (jnp.dot(x_ref[...], ws_ref[...], preferred_element_type=jnp.float32)
         + jnp.dot(hn, wn_ref[...], preferred_element_type=jnp.float32)
         + b_ref[...])
    z_ref[...] = z
    zs = jnp.sum(z, axis=0, keepdims=True)
    z2 = jnp.sum(z * z, axis=0, keepdims=True)

    @pl.when(i == 0)
    def _():
        sum_ref[...] = zs
        sq_ref[...] = z2

    @pl.when(i > 0)
    def _():
        sum_ref[...] += zs
        sq_ref[...] += z2


def _dense1(x, sa, sb, deg_col, ws, wn, b):
    d_in = x.shape[1]
    return pl.pallas_call(
        _dense1_body,
        grid=(GRID,),
        in_specs=[
            pl.BlockSpec((RB, d_in), lambda i: (i, 0)),
            pl.BlockSpec((RB, d_in), lambda i: (i, 0)),
            pl.BlockSpec((RB, d_in), lambda i: (i, 0)),
            pl.BlockSpec((RB, 1), lambda i: (i, 0)),
            pl.BlockSpec((d_in, D_H), lambda i: (0, 0)),
            pl.BlockSpec((d_in, D_H), lambda i: (0, 0)),
            pl.BlockSpec((1, D_H), lambda i: (0, 0)),
        ],
        out_specs=[
            pl.BlockSpec((RB, D_H), lambda i: (i, 0)),
            pl.BlockSpec((1, D_H), lambda i: (0, 0)),
            pl.BlockSpec((1, D_H), lambda i: (0, 0)),
        ],
        out_shape=[
            jax.ShapeDtypeStruct((N, D_H), jnp.float32),
            jax.ShapeDtypeStruct((1, D_H), jnp.float32),
            jax.ShapeDtypeStruct((1, D_H), jnp.float32),
        ],
    )(x, sa, sb, deg_col, ws, wn, b)


def _dense2_body(ha_ref, hb_ref, sa_ref, sb_ref, deg_ref, ws_ref, wn_ref,
                 b_ref, z_ref, sum_ref, sq_ref):
    i = pl.program_id(0)
    r = 1.0 / jnp.maximum(deg_ref[...], 1.0)
    hna = sa_ref[...] * r
    hnb = sb_ref[...] * r
    ws = ws_ref[...]
    wn = wn_ref[...]
    z = (jnp.dot(ha_ref[...], ws[:D_IN, :], preferred_element_type=jnp.float32)
         + jnp.dot(hb_ref[...], ws[D_IN:, :], preferred_element_type=jnp.float32)
         + jnp.dot(hna, wn[:D_IN, :], preferred_element_type=jnp.float32)
         + jnp.dot(hnb, wn[D_IN:, :], preferred_element_type=jnp.float32)
         + b_ref[...])
    z_ref[...] = z
    zs = jnp.sum(z, axis=0, keepdims=True)
    z2 = jnp.sum(z * z, axis=0, keepdims=True)

    @pl.when(i == 0)
    def _():
        sum_ref[...] = zs
        sq_ref[...] = z2

    @pl.when(i > 0)
    def _():
        sum_ref[...] += zs
        sq_ref[...] += z2


def _dense2(ha, hb, sa, sb, deg_col, ws, wn, b):
    return pl.pallas_call(
        _dense2_body,
        grid=(GRID,),
        in_specs=[
            pl.BlockSpec((RB, D_IN), lambda i: (i, 0)),
            pl.BlockSpec((RB, D_IN), lambda i: (i, 0)),
            pl.BlockSpec((RB, D_IN), lambda i: (i, 0)),
            pl.BlockSpec((RB, D_IN), lambda i: (i, 0)),
            pl.BlockSpec((RB, 1), lambda i: (i, 0)),
            pl.BlockSpec((D_H, D_H), lambda i: (0, 0)),
            pl.BlockSpec((D_H, D_H), lambda i: (0, 0)),
            pl.BlockSpec((1, D_H), lambda i: (0, 0)),
        ],
        out_specs=[
            pl.BlockSpec((RB, D_H), lambda i: (i, 0)),
            pl.BlockSpec((1, D_H), lambda i: (0, 0)),
            pl.BlockSpec((1, D_H), lambda i: (0, 0)),
        ],
        out_shape=[
            jax.ShapeDtypeStruct((N, D_H), jnp.float32),
            jax.ShapeDtypeStruct((1, D_H), jnp.float32),
            jax.ShapeDtypeStruct((1, D_H), jnp.float32),
        ],
    )(ha, hb, sa, sb, deg_col, ws, wn, b)


def _bnrelu_split_body(z_ref, sum_ref, sq_ref, g_ref, be_ref, ha_ref, hb_ref):
    mu = sum_ref[...] * (1.0 / N)
    var = sq_ref[...] * (1.0 / N) - mu * mu
    inv = lax.rsqrt(var + 1e-5) * g_ref[...]
    h = jnp.maximum((z_ref[...] - mu) * inv + be_ref[...], 0.0)
    ha_ref[...] = h[:, :D_IN]
    hb_ref[...] = h[:, D_IN:]


def _bnrelu_split(z, zsum, zsq, gamma, beta):
    return pl.pallas_call(
        _bnrelu_split_body,
        grid=(GRID,),
        in_specs=[
            pl.BlockSpec((RB, D_H), lambda i: (i, 0)),
            pl.BlockSpec((1, D_H), lambda i: (0, 0)),
            pl.BlockSpec((1, D_H), lambda i: (0, 0)),
            pl.BlockSpec((1, D_H), lambda i: (0, 0)),
            pl.BlockSpec((1, D_H), lambda i: (0, 0)),
        ],
        out_specs=[
            pl.BlockSpec((RB, D_IN), lambda i: (i, 0)),
            pl.BlockSpec((RB, D_IN), lambda i: (i, 0)),
        ],
        out_shape=[
            jax.ShapeDtypeStruct((N_ROWS, D_IN), jnp.float32),
            jax.ShapeDtypeStruct((N_ROWS, D_IN), jnp.float32),
        ],
    )(z, zsum, zsq, gamma, beta)


def _final_body(z_ref, sum_ref, sq_ref, g_ref, be_ref, c_ref,
                ws2_ref, wn2_ref, b2_ref, wlin_ref, blin_ref,
                out_ref, s2_acc, t2_acc):
    i = pl.program_id(0)
    mu = sum_ref[...] * (1.0 / N)
    var = sq_ref[...] * (1.0 / N) - mu * mu
    inv = lax.rsqrt(var + 1e-5) * g_ref[...]
    h = jnp.maximum((z_ref[...] - mu) * inv + be_ref[...], 0.0)
    s2 = jnp.sum(h, axis=0, keepdims=True)
    t2 = jnp.sum(h * c_ref[...], axis=0, keepdims=True)

    @pl.when(i == 0)
    def _():
        s2_acc[...] = s2
        t2_acc[...] = t2

    @pl.when(i > 0)
    def _():
        s2_acc[...] += s2
        t2_acc[...] += t2

    @pl.when(i == GRID - 1)
    def _():
        hg = (jnp.dot(s2_acc[...], ws2_ref[...], preferred_element_type=jnp.float32)
              + jnp.dot(t2_acc[...], wn2_ref[...], preferred_element_type=jnp.float32)
              + float(N) * b2_ref[...])
        out_ref[...] = (jnp.dot(hg, wlin_ref[...], preferred_element_type=jnp.float32)
                        + blin_ref[...])


def _final(z, zsum, zsq, gamma, beta, c_col, ws2, wn2, b2, wlin, blin):
    return pl.pallas_call(
        _final_body,
        grid=(GRID,),
        in_specs=[
            pl.BlockSpec((RB, D_H), lambda i: (i, 0)),
            pl.BlockSpec((1, D_H), lambda i: (0, 0)),
            pl.BlockSpec((1, D_H), lambda i: (0, 0)),
            pl.BlockSpec((1, D_H), lambda i: (0, 0)),
            pl.BlockSpec((1, D_H), lambda i: (0, 0)),
            pl.BlockSpec((RB, 1), lambda i: (i, 0)),
            pl.BlockSpec((D_H, D_H), lambda i: (0, 0)),
            pl.BlockSpec((D_H, D_H), lambda i: (0, 0)),
            pl.BlockSpec((1, D_H), lambda i: (0, 0)),
            pl.BlockSpec((D_H, D_OUT), lambda i: (0, 0)),
            pl.BlockSpec((1, D_OUT), lambda i: (0, 0)),
        ],
        out_specs=pl.BlockSpec((1, D_OUT), lambda i: (0, 0)),
        out_shape=jax.ShapeDtypeStruct((1, D_OUT), jnp.float32),
        scratch_shapes=[
            pltpu.VMEM((1, D_H), jnp.float32),
            pltpu.VMEM((1, D_H), jnp.float32),
        ],
    )(z, zsum, zsq, gamma, beta, c_col, ws2, wn2, b2, wlin, blin)


def kernel(x, edge_index, W_self0, W_neigh0, b0, W_self1, W_neigh1, b1,
           W_self2, W_neigh2, b2, gamma0, beta0, gamma1, beta1, W_lin, b_lin):
    src = edge_index[0]
    dst = edge_index[1]
    pad = E_PAD - E
    srcm = jnp.concatenate([src, jnp.zeros((pad,), jnp.int32)]).reshape(EROWS, K)
    dstm = jnp.concatenate([dst, jnp.full((pad,), N, jnp.int32)]).reshape(EROWS, K)
    zfeat = jnp.zeros((ROWS_PER_TILE, D_IN), jnp.float32)
    zdeg = jnp.zeros((DEG_PER_TILE,), jnp.float32)

    s0a, s0b, dega, degb = _agg1(x, srcm, dstm, zfeat, zdeg)
    deg_col = (dega + degb)[:, None]

    z1, z1s, z1q = _dense1(x, s0a, s0b, deg_col, W_self0, W_neigh0,
                           b0[None, :])
    h1a, h1b = _bnrelu_split(z1, z1s, z1q, gamma0[None, :], beta0[None, :])

    s1a, s1b, cvec = _agg2(h1a, h1b, dega + degb, srcm, dstm, zfeat, zdeg)
    c_col = cvec[:, None]

    z2, z2s, z2q = _dense2(h1a, h1b, s1a, s1b, deg_col,
                           W_self1, W_neigh1, b1[None, :])
    out = _final(z2, z2s, z2q, gamma1[None, :], beta1[None, :], c_col,
                 W_self2, W_neigh2, b2[None, :], W_lin, b_lin[None, :])
    return out


# speedup vs baseline: 4.4955x; 1.2311x over previous
import functools

import jax
import jax.numpy as jnp
from jax import lax
from jax.experimental import pallas as pl
from jax.experimental.pallas import tpu as pltpu
from jax.experimental.pallas import tpu_sc as plsc

N = 10000
E = 320000
D_IN = 128
D_H = 256
D_OUT = 64

NC = 2
NS = 16
K = 128
EROWS = 2560
E_PAD = EROWS * K
N_ROWS = 10240
ROWS_PER_TILE = N_ROWS // NS
N_DEG = 10240
DEG_PER_TILE = N_DEG // NS
A_ROWS = EROWS // (NC * NS)
C_ROWS = EROWS // NS
IB = 16

_mesh = plsc.VectorSubcoreMesh(core_axis_name="c", subcore_axis_name="s")


def _zero_accs(zfeat, zdeg, acc, dacc, s):
    pltpu.sync_copy(zfeat, acc.at[pl.ds(s * ROWS_PER_TILE, ROWS_PER_TILE)])
    pltpu.sync_copy(zdeg, dacc.at[pl.ds(s * DEG_PER_TILE, DEG_PER_TILE)])


def _fill_ones(ones_v):
    for j in range(K // 16):
        ones_v[pl.ds(j * 16, 16)] = jnp.ones((16,), jnp.float32)


def _agg1_body(x_hbm, srcm, dstm, zfeat, zdeg,
               s0a, s0b, dega, degb,
               src_v, dst_v, rows0, rows1, ones_v,
               sem_g0, sem_g1, sem_s0, sem_s1, sem_d,
               acc, dacc):
    c = lax.axis_index("c")
    s = lax.axis_index("s")
    wid = c * NS + s
    _zero_accs(zfeat, zdeg, acc, dacc, s)
    base = wid * A_ROWS
    _fill_ones(ones_v)
    plsc.subcore_barrier()

    def blk(bi, carry):
        pltpu.sync_copy(srcm.at[pl.ds(base + bi * IB, IB)], src_v)
        pltpu.sync_copy(dstm.at[pl.ds(base + bi * IB, IB)], dst_v)
        pltpu.async_copy(x_hbm.at[src_v.at[0]], rows0, sem_g0)

        def pair(t, carry2):
            j0 = 2 * t
            j1 = j0 + 1
            @pl.when(t > 0)
            def _():
                pltpu.make_async_copy(rows1, acc.at[dst_v.at[j1]], sem_s1).wait()
                pltpu.make_async_copy(ones_v, dacc.at[dst_v.at[j0]], sem_d).wait()
                pltpu.make_async_copy(ones_v, dacc.at[dst_v.at[j1]], sem_d).wait()

            pltpu.async_copy(x_hbm.at[src_v.at[j1]], rows1, sem_g1)
            pltpu.make_async_copy(x_hbm.at[src_v.at[j0]], rows0, sem_g0).wait()
            pltpu.async_copy(rows0, acc.at[dst_v.at[j0]], sem_s0, add=True)
            pltpu.async_copy(ones_v, dacc.at[dst_v.at[j0]], sem_d, add=True)
            pltpu.async_copy(ones_v, dacc.at[dst_v.at[j1]], sem_d, add=True)
            pltpu.make_async_copy(x_hbm.at[src_v.at[j1]], rows1, sem_g1).wait()
            pltpu.async_copy(rows1, acc.at[dst_v.at[j1]], sem_s1, add=True)

            @pl.when(t < IB // 2 - 1)
            def _():
                pltpu.make_async_copy(rows0, acc.at[dst_v.at[j0]], sem_s0).wait()
                pltpu.async_copy(x_hbm.at[src_v.at[j0 + 2]], rows0, sem_g0)

            return carry2

        lax.fori_loop(0, IB // 2, pair, carry)
        pltpu.make_async_copy(rows0, acc.at[dst_v.at[0]], sem_s0).wait()
        pltpu.make_async_copy(rows1, acc.at[dst_v.at[0]], sem_s1).wait()
        pltpu.make_async_copy(ones_v, dacc.at[dst_v.at[0]], sem_d).wait()
        pltpu.make_async_copy(ones_v, dacc.at[dst_v.at[0]], sem_d).wait()
        return carry

    lax.fori_loop(0, A_ROWS // IB, blk, 0)
    plsc.subcore_barrier()

    rsl = pl.ds(s * ROWS_PER_TILE, ROWS_PER_TILE)
    dsl = pl.ds(s * DEG_PER_TILE, DEG_PER_TILE)

    @pl.when(c == 0)
    def _():
        pltpu.sync_copy(acc.at[rsl], s0a.at[rsl])
        pltpu.sync_copy(dacc.at[dsl], dega.at[dsl])

    @pl.when(c == 1)
    def _():
        pltpu.sync_copy(acc.at[rsl], s0b.at[rsl])
        pltpu.sync_copy(dacc.at[dsl], degb.at[dsl])


_agg1 = functools.partial(
    pl.kernel,
    out_type=[
        jax.ShapeDtypeStruct((N_ROWS, D_IN), jnp.float32),
        jax.ShapeDtypeStruct((N_ROWS, D_IN), jnp.float32),
        jax.ShapeDtypeStruct((N_DEG,), jnp.float32),
        jax.ShapeDtypeStruct((N_DEG,), jnp.float32),
    ],
    mesh=_mesh,
    scratch_types=[
        pltpu.VMEM((IB, K), jnp.int32),
        pltpu.VMEM((IB, K), jnp.int32),
        pltpu.VMEM((K, D_IN), jnp.float32),
        pltpu.VMEM((K, D_IN), jnp.float32),
        pltpu.VMEM((K,), jnp.float32),
        pltpu.SemaphoreType.DMA,
        pltpu.SemaphoreType.DMA,
        pltpu.SemaphoreType.DMA,
        pltpu.SemaphoreType.DMA,
        pltpu.SemaphoreType.DMA,
        pltpu.VMEM_SHARED((N_ROWS, D_IN), jnp.float32),
        pltpu.VMEM_SHARED((N_DEG,), jnp.float32),
    ],
)(_agg1_body)


def _agg2_body(h1a, h1b, deg_hbm, srcm, dstm, zfeat, zdeg,
               s1a, s1b, couta, coutb,
               src_v, dst_v, rows0, rows1, degv0, degv1, wv0, wv1,
               sem_g0, sem_g1, sem_s0, sem_s1, sem_c0, sem_c1, sem_w0, sem_w1,
               acc, cacc):
    c = lax.axis_index("c")
    s = lax.axis_index("s")
    _zero_accs(zfeat, zdeg, acc, cacc, s)
    base = s * C_ROWS
    plsc.subcore_barrier()

    def gather_feat(j, buf, sem):
        @pl.when(c == 0)
        def _():
            pltpu.async_copy(h1a.at[src_v.at[j]], buf, sem)

        @pl.when(c == 1)
        def _():
            pltpu.async_copy(h1b.at[src_v.at[j]], buf, sem)

    def wait_feat(j, buf, sem):
        pltpu.make_async_copy(h1a.at[src_v.at[j]], buf, sem).wait()

    def c_half(do_c, t, bi, j, degv, wv, sem_c, sem_w):
        @pl.when(do_c)
        def _():
            @pl.when(t > 0)
            def _():
                pltpu.make_async_copy(wv, cacc.at[src_v.at[j]], sem_w).wait()

            pltpu.make_async_copy(deg_hbm.at[dst_v.at[j]], degv, sem_c).wait()
            eid0 = (base + bi * IB + j) * K
            for u in range(K // 16):
                d16 = degv[pl.ds(u * 16, 16)]
                eid = eid0 + u * 16 + lax.iota(jnp.int32, 16)
                w16 = jnp.where(eid < E, 1.0 / jnp.maximum(d16, 1.0), 0.0)
                wv[pl.ds(u * 16, 16)] = w16
            pltpu.async_copy(wv, cacc.at[src_v.at[j]], sem_w, add=True)

    def blk(bi, carry):
        do_c = (c == 0) == (bi < (C_ROWS // IB) // 2)
        pltpu.sync_copy(srcm.at[pl.ds(base + bi * IB, IB)], src_v)
        pltpu.sync_copy(dstm.at[pl.ds(base + bi * IB, IB)], dst_v)
        gather_feat(0, rows0, sem_g0)

        def pair(t, carry2):
            j0 = 2 * t
            j1 = j0 + 1

            @pl.when(t > 0)
            def _():
                pltpu.make_async_copy(rows1, acc.at[dst_v.at[j1]], sem_s1).wait()

            gather_feat(j1, rows1, sem_g1)

            @pl.when(do_c)
            def _():
                pltpu.async_copy(deg_hbm.at[dst_v.at[j0]], degv0, sem_c0)
                pltpu.async_copy(deg_hbm.at[dst_v.at[j1]], degv1, sem_c1)

            wait_feat(j0, rows0, sem_g0)
            pltpu.async_copy(rows0, acc.at[dst_v.at[j0]], sem_s0, add=True)
            c_half(do_c, t, bi, j0, degv0, wv0, sem_c0, sem_w0)
            wait_feat(j1, rows1, sem_g1)
            pltpu.async_copy(rows1, acc.at[dst_v.at[j1]], sem_s1, add=True)
            c_half(do_c, t, bi, j1, degv1, wv1, sem_c1, sem_w1)

            @pl.when(t < IB // 2 - 1)
            def _():
                pltpu.make_async_copy(rows0, acc.at[dst_v.at[j0]], sem_s0).wait()
                gather_feat(j0 + 2, rows0, sem_g0)

            return carry2

        lax.fori_loop(0, IB // 2, pair, carry)
        pltpu.make_async_copy(rows0, acc.at[dst_v.at[0]], sem_s0).wait()
        pltpu.make_async_copy(rows1, acc.at[dst_v.at[0]], sem_s1).wait()

        @pl.when(do_c)
        def _():
            pltpu.make_async_copy(wv0, cacc.at[src_v.at[0]], sem_w0).wait()
            pltpu.make_async_copy(wv1, cacc.at[src_v.at[0]], sem_w1).wait()

        return carry

    lax.fori_loop(0, C_ROWS // IB, blk, 0)
    plsc.subcore_barrier()

    rsl = pl.ds(s * ROWS_PER_TILE, ROWS_PER_TILE)
    dsl = pl.ds(s * DEG_PER_TILE, DEG_PER_TILE)

    @pl.when(c == 0)
    def _():
        pltpu.sync_copy(acc.at[rsl], s1a.at[rsl])
        pltpu.sync_copy(cacc.at[dsl], couta.at[dsl])

    @pl.when(c == 1)
    def _():
        pltpu.sync_copy(acc.at[rsl], s1b.at[rsl])
        pltpu.sync_copy(cacc.at[dsl], coutb.at[dsl])


_agg2 = functools.partial(
    pl.kernel,
    out_type=[
        jax.ShapeDtypeStruct((N_ROWS, D_IN), jnp.float32),
        jax.ShapeDtypeStruct((N_ROWS, D_IN), jnp.float32),
        jax.ShapeDtypeStruct((N_DEG,), jnp.float32),
        jax.ShapeDtypeStruct((N_DEG,), jnp.float32),
    ],
    mesh=_mesh,
    scratch_types=[
        pltpu.VMEM((IB, K), jnp.int32),
        pltpu.VMEM((IB, K), jnp.int32),
        pltpu.VMEM((K, D_IN), jnp.float32),
        pltpu.VMEM((K, D_IN), jnp.float32),
        pltpu.VMEM((K,), jnp.float32),
        pltpu.VMEM((K,), jnp.float32),
        pltpu.VMEM((K,), jnp.float32),
        pltpu.VMEM((K,), jnp.float32),
        pltpu.SemaphoreType.DMA,
        pltpu.SemaphoreType.DMA,
        pltpu.SemaphoreType.DMA,
        pltpu.SemaphoreType.DMA,
        pltpu.SemaphoreType.DMA,
        pltpu.SemaphoreType.DMA,
        pltpu.SemaphoreType.DMA,
        pltpu.SemaphoreType.DMA,
        pltpu.VMEM_SHARED((N_ROWS, D_IN), jnp.float32),
        pltpu.VMEM_SHARED((N_DEG,), jnp.float32),
    ],
)(_agg2_body)


RB = 400
GRID = N // RB


def _dense1_body(x_ref, sa_ref, sb_ref, deg_ref, ws_ref, wn_ref, b_ref,
                 z_ref, sum_ref, sq_ref):
    i = pl.program_id(0)
    r = 1.0 / jnp.maximum(deg_ref[...], 1.0)
    hn = (sa_ref[...] + sb_ref[...]) * r
    z = (jnp.dot(x_ref[...], ws_ref[...], preferred_element_type=jnp.float32)
         + jnp.dot(hn, wn_ref[...], preferred_element_type=jnp.float32)
         + b_ref[...])
    z_ref[...] = z
    zs = jnp.sum(z, axis=0, keepdims=True)
    z2 = jnp.sum(z * z, axis=0, keepdims=True)

    @pl.when(i == 0)
    def _():
        sum_ref[...] = zs
        sq_ref[...] = z2

    @pl.when(i > 0)
    def _():
        sum_ref[...] += zs
        sq_ref[...] += z2


def _dense1(x, sa, sb, deg_col, ws, wn, b):
    d_in = x.shape[1]
    return pl.pallas_call(
        _dense1_body,
        grid=(GRID,),
        in_specs=[
            pl.BlockSpec((RB, d_in), lambda i: (i, 0)),
            pl.BlockSpec((RB, d_in), lambda i: (i, 0)),
            pl.BlockSpec((RB, d_in), lambda i: (i, 0)),
            pl.BlockSpec((RB, 1), lambda i: (i, 0)),
            pl.BlockSpec((d_in, D_H), lambda i: (0, 0)),
            pl.BlockSpec((d_in, D_H), lambda i: (0, 0)),
            pl.BlockSpec((1, D_H), lambda i: (0, 0)),
        ],
        out_specs=[
            pl.BlockSpec((RB, D_H), lambda i: (i, 0)),
            pl.BlockSpec((1, D_H), lambda i: (0, 0)),
            pl.BlockSpec((1, D_H), lambda i: (0, 0)),
        ],
        out_shape=[
            jax.ShapeDtypeStruct((N, D_H), jnp.float32),
            jax.ShapeDtypeStruct((1, D_H), jnp.float32),
            jax.ShapeDtypeStruct((1, D_H), jnp.float32),
        ],
    )(x, sa, sb, deg_col, ws, wn, b)


def _dense2_body(ha_ref, hb_ref, sa_ref, sb_ref, deg_ref, ws_ref, wn_ref,
                 b_ref, z_ref, sum_ref, sq_ref):
    i = pl.program_id(0)
    r = 1.0 / jnp.maximum(deg_ref[...], 1.0)
    hna = sa_ref[...] * r
    hnb = sb_ref[...] * r
    ws = ws_ref[...]
    wn = wn_ref[...]
    z = (jnp.dot(ha_ref[...], ws[:D_IN, :], preferred_element_type=jnp.float32)
         + jnp.dot(hb_ref[...], ws[D_IN:, :], preferred_element_type=jnp.float32)
         + jnp.dot(hna, wn[:D_IN, :], preferred_element_type=jnp.float32)
         + jnp.dot(hnb, wn[D_IN:, :], preferred_element_type=jnp.float32)
         + b_ref[...])
    z_ref[...] = z
    zs = jnp.sum(z, axis=0, keepdims=True)
    z2 = jnp.sum(z * z, axis=0, keepdims=True)

    @pl.when(i == 0)
    def _():
        sum_ref[...] = zs
        sq_ref[...] = z2

    @pl.when(i > 0)
    def _():
        sum_ref[...] += zs
        sq_ref[...] += z2


def _dense2(ha, hb, sa, sb, deg_col, ws, wn, b):
    return pl.pallas_call(
        _dense2_body,
        grid=(GRID,),
        in_specs=[
            pl.BlockSpec((RB, D_IN), lambda i: (i, 0)),
            pl.BlockSpec((RB, D_IN), lambda i: (i, 0)),
            pl.BlockSpec((RB, D_IN), lambda i: (i, 0)),
            pl.BlockSpec((RB, D_IN), lambda i: (i, 0)),
            pl.BlockSpec((RB, 1), lambda i: (i, 0)),
            pl.BlockSpec((D_H, D_H), lambda i: (0, 0)),
            pl.BlockSpec((D_H, D_H), lambda i: (0, 0)),
            pl.BlockSpec((1, D_H), lambda i: (0, 0)),
        ],
        out_specs=[
            pl.BlockSpec((RB, D_H), lambda i: (i, 0)),
            pl.BlockSpec((1, D_H), lambda i: (0, 0)),
            pl.BlockSpec((1, D_H), lambda i: (0, 0)),
        ],
        out_shape=[
            jax.ShapeDtypeStruct((N, D_H), jnp.float32),
            jax.ShapeDtypeStruct((1, D_H), jnp.float32),
            jax.ShapeDtypeStruct((1, D_H), jnp.float32),
        ],
    )(ha, hb, sa, sb, deg_col, ws, wn, b)


def _bnrelu_split_body(z_ref, sum_ref, sq_ref, g_ref, be_ref, ha_ref, hb_ref):
    mu = sum_ref[...] * (1.0 / N)
    var = sq_ref[...] * (1.0 / N) - mu * mu
    inv = lax.rsqrt(var + 1e-5) * g_ref[...]
    h = jnp.maximum((z_ref[...] - mu) * inv + be_ref[...], 0.0)
    ha_ref[...] = h[:, :D_IN]
    hb_ref[...] = h[:, D_IN:]


def _bnrelu_split(z, zsum, zsq, gamma, beta):
    return pl.pallas_call(
        _bnrelu_split_body,
        grid=(GRID,),
        in_specs=[
            pl.BlockSpec((RB, D_H), lambda i: (i, 0)),
            pl.BlockSpec((1, D_H), lambda i: (0, 0)),
            pl.BlockSpec((1, D_H), lambda i: (0, 0)),
            pl.BlockSpec((1, D_H), lambda i: (0, 0)),
            pl.BlockSpec((1, D_H), lambda i: (0, 0)),
        ],
        out_specs=[
            pl.BlockSpec((RB, D_IN), lambda i: (i, 0)),
            pl.BlockSpec((RB, D_IN), lambda i: (i, 0)),
        ],
        out_shape=[
            jax.ShapeDtypeStruct((N_ROWS, D_IN), jnp.float32),
            jax.ShapeDtypeStruct((N_ROWS, D_IN), jnp.float32),
        ],
    )(z, zsum, zsq, gamma, beta)


def _final_body(z_ref, sum_ref, sq_ref, g_ref, be_ref, c_ref,
                ws2_ref, wn2_ref, b2_ref, wlin_ref, blin_ref,
                out_ref, s2_acc, t2_acc):
    i = pl.program_id(0)
    mu = sum_ref[...] * (1.0 / N)
    var = sq_ref[...] * (1.0 / N) - mu * mu
    inv = lax.rsqrt(var + 1e-5) * g_ref[...]
    h = jnp.maximum((z_ref[...] - mu) * inv + be_ref[...], 0.0)
    s2 = jnp.sum(h, axis=0, keepdims=True)
    t2 = jnp.sum(h * c_ref[...], axis=0, keepdims=True)

    @pl.when(i == 0)
    def _():
        s2_acc[...] = s2
        t2_acc[...] = t2

    @pl.when(i > 0)
    def _():
        s2_acc[...] += s2
        t2_acc[...] += t2

    @pl.when(i == GRID - 1)
    def _():
        hg = (jnp.dot(s2_acc[...], ws2_ref[...], preferred_element_type=jnp.float32)
              + jnp.dot(t2_acc[...], wn2_ref[...], preferred_element_type=jnp.float32)
              + float(N) * b2_ref[...])
        out_ref[...] = (jnp.dot(hg, wlin_ref[...], preferred_element_type=jnp.float32)
                        + blin_ref[...])


def _final(z, zsum, zsq, gamma, beta, c_col, ws2, wn2, b2, wlin, blin):
    return pl.pallas_call(
        _final_body,
        grid=(GRID,),
        in_specs=[
            pl.BlockSpec((RB, D_H), lambda i: (i, 0)),
            pl.BlockSpec((1, D_H), lambda i: (0, 0)),
            pl.BlockSpec((1, D_H), lambda i: (0, 0)),
            pl.BlockSpec((1, D_H), lambda i: (0, 0)),
            pl.BlockSpec((1, D_H), lambda i: (0, 0)),
            pl.BlockSpec((RB, 1), lambda i: (i, 0)),
            pl.BlockSpec((D_H, D_H), lambda i: (0, 0)),
            pl.BlockSpec((D_H, D_H), lambda i: (0, 0)),
            pl.BlockSpec((1, D_H), lambda i: (0, 0)),
            pl.BlockSpec((D_H, D_OUT), lambda i: (0, 0)),
            pl.BlockSpec((1, D_OUT), lambda i: (0, 0)),
        ],
        out_specs=pl.BlockSpec((1, D_OUT), lambda i: (0, 0)),
        out_shape=jax.ShapeDtypeStruct((1, D_OUT), jnp.float32),
        scratch_shapes=[
            pltpu.VMEM((1, D_H), jnp.float32),
            pltpu.VMEM((1, D_H), jnp.float32),
        ],
    )(z, zsum, zsq, gamma, beta, c_col, ws2, wn2, b2, wlin, blin)


def kernel(x, edge_index, W_self0, W_neigh0, b0, W_self1, W_neigh1, b1,
           W_self2, W_neigh2, b2, gamma0, beta0, gamma1, beta1, W_lin, b_lin):
    src = edge_index[0]
    dst = edge_index[1]
    pad = E_PAD - E
    srcm = jnp.concatenate([src, jnp.zeros((pad,), jnp.int32)]).reshape(EROWS, K)
    dstm = jnp.concatenate([dst, jnp.full((pad,), N, jnp.int32)]).reshape(EROWS, K)
    zfeat = jnp.zeros((ROWS_PER_TILE, D_IN), jnp.float32)
    zdeg = jnp.zeros((DEG_PER_TILE,), jnp.float32)

    s0a, s0b, dega, degb = _agg1(x, srcm, dstm, zfeat, zdeg)
    deg_col = (dega + degb)[:, None]

    z1, z1s, z1q = _dense1(x, s0a, s0b, deg_col, W_self0, W_neigh0,
                           b0[None, :])
    h1a, h1b = _bnrelu_split(z1, z1s, z1q, gamma0[None, :], beta0[None, :])

    s1a, s1b, couta, coutb = _agg2(h1a, h1b, dega + degb, srcm, dstm, zfeat, zdeg)
    c_col = (couta + coutb)[:, None]

    z2, z2s, z2q = _dense2(h1a, h1b, s1a, s1b, deg_col,
                           W_self1, W_neigh1, b1[None, :])
    out = _final(z2, z2s, z2q, gamma1[None, :], beta1[None, :], c_col,
                 W_self2, W_neigh2, b2[None, :], W_lin, b_lin[None, :])
    return out
